# baseline (speedup 1.0000x reference)
import functools
import math

import jax
import jax.numpy as jnp
from jax import lax
from jax.experimental import pallas as pl
from jax.experimental.pallas import tpu as pltpu
from jax.experimental.pallas import tpu_sc as plsc

VOCAB = 1000000
EMB = 64
B = 16384
L = 50
SCALE = math.sqrt(EMB)

_NC = 2
_NS = 16
_NW = _NC * _NS
_N_IDX = B * L
_PER_W = _N_IDX // _NW
_CHUNK = 512
_SUB = 128
_N_CHUNKS = _PER_W // _CHUNK


def _make_lookup():
    mesh = plsc.VectorSubcoreMesh(core_axis_name="c", subcore_axis_name="s")

    @functools.partial(
        pl.kernel,
        mesh=mesh,
        out_type=jax.ShapeDtypeStruct((_N_IDX, EMB), jnp.float32),
        scratch_types=[
            pltpu.VMEM((_CHUNK,), jnp.int32),
            pltpu.VMEM((_CHUNK, EMB), jnp.float32),
            pltpu.SemaphoreType.DMA,
        ],
        compiler_params=pltpu.CompilerParams(use_tc_tiling_on_sc=False),
    )
    def lookup(tok_hbm, table_hbm, out_hbm, idx_v, rows_v, sem):
        wid = lax.axis_index("s") * _NC + lax.axis_index("c")
        base = wid * _PER_W

        def chunk_body(g, _):
            off = base + g * _CHUNK
            pltpu.sync_copy(tok_hbm.at[pl.ds(off, _CHUNK)], idx_v)
            copies = []
            for j in range(_CHUNK // _SUB):
                copies.append(pltpu.async_copy(
                    table_hbm.at[idx_v.at[pl.ds(j * _SUB, _SUB)]],
                    rows_v.at[pl.ds(j * _SUB, _SUB)],
                    sem,
                ))
            for c in copies:
                c.wait()

            def scale_body(r, _):
                for c in range(EMB // 16):
                    v = rows_v[r, pl.ds(c * 16, 16)]
                    rows_v[r, pl.ds(c * 16, 16)] = v * SCALE
                return 0

            lax.fori_loop(0, _CHUNK, scale_body, 0)
            pltpu.sync_copy(rows_v, out_hbm.at[pl.ds(off, _CHUNK)])
            return 0

        lax.fori_loop(0, _N_CHUNKS, chunk_body, 0)

    return lookup


_lookup = _make_lookup()


def kernel(tokens, table):
    tok_flat = tokens.reshape(_N_IDX)
    out = _lookup(tok_flat, table)
    return out.reshape(B, L, EMB)


# speedup vs baseline: 1.1172x; 1.1172x over previous
import functools
import math

import jax
import jax.numpy as jnp
from jax import lax
from jax.experimental import pallas as pl
from jax.experimental.pallas import tpu as pltpu
from jax.experimental.pallas import tpu_sc as plsc

VOCAB = 1000000
EMB = 64
B = 16384
L = 50
SCALE = math.sqrt(EMB)

_NC = 2
_NS = 16
_NW = _NC * _NS
_N_IDX = B * L
_PER_W = _N_IDX // _NW
_CHUNK = 256
_SUB = 128
_NSUB = _CHUNK // _SUB
_N_CHUNKS = _PER_W // _CHUNK
_UR = 4


def _make_lookup():
    mesh = plsc.VectorSubcoreMesh(core_axis_name="c", subcore_axis_name="s")

    @functools.partial(
        pl.kernel,
        mesh=mesh,
        out_type=jax.ShapeDtypeStruct((_N_IDX, EMB), jnp.float32),
        scratch_types=[
            pltpu.VMEM((2, _CHUNK), jnp.int32),
            pltpu.VMEM((2, _CHUNK, EMB), jnp.float32),
            pltpu.VMEM((2, _CHUNK, EMB), jnp.float32),
            pltpu.SemaphoreType.DMA,
            pltpu.SemaphoreType.DMA,
            pltpu.SemaphoreType.DMA,
            pltpu.SemaphoreType.DMA,
            pltpu.SemaphoreType.DMA,
            pltpu.SemaphoreType.DMA,
        ],
        compiler_params=pltpu.CompilerParams(use_tc_tiling_on_sc=False),
    )
    def lookup(tok_hbm, table_hbm, out_hbm, idx_v, rows_in, rows_out,
               g0, g1, i0, i1, w0, w1):
        gsems = (g0, g1)
        isems = (i0, i1)
        wsems = (w0, w1)
        wid = lax.axis_index("s") * _NC + lax.axis_index("c")
        base = wid * _PER_W

        def issue_idx(g, s):
            pltpu.async_copy(
                tok_hbm.at[pl.ds(base + g * _CHUNK, _CHUNK)],
                idx_v.at[s], isems[s])

        def wait_idx(s):
            pltpu.make_async_copy(
                tok_hbm.at[pl.ds(base, _CHUNK)],
                idx_v.at[s], isems[s]).wait()

        def issue_gather(s):
            for j in range(_NSUB):
                pltpu.async_copy(
                    table_hbm.at[idx_v.at[s].at[pl.ds(j * _SUB, _SUB)]],
                    rows_in.at[s].at[pl.ds(j * _SUB, _SUB)],
                    gsems[s])

        def wait_gather(s):
            for j in range(_NSUB):
                pltpu.make_async_copy(
                    table_hbm.at[idx_v.at[s].at[pl.ds(j * _SUB, _SUB)]],
                    rows_in.at[s].at[pl.ds(j * _SUB, _SUB)],
                    gsems[s]).wait()

        def issue_wb(g, s):
            pltpu.async_copy(
                rows_out.at[s],
                out_hbm.at[pl.ds(base + g * _CHUNK, _CHUNK)], wsems[s])

        def wait_wb(s):
            pltpu.make_async_copy(
                rows_out.at[s],
                out_hbm.at[pl.ds(base, _CHUNK)], wsems[s]).wait()

        def scale(s):
            def body(r4, _):
                r = r4 * _UR
                for dr in range(_UR):
                    for c in range(EMB // 16):
                        v = rows_in[s, r + dr, pl.ds(c * 16, 16)]
                        rows_out[s, r + dr, pl.ds(c * 16, 16)] = v * SCALE
                return 0
            lax.fori_loop(0, _CHUNK // _UR, body, 0)

        def step(g, s, wait_w, iss_idx, iss_gather):
            wait_gather(s)
            if iss_idx:
                issue_idx(g + 2, s)
            if iss_gather:
                wait_idx(1 - s)
                issue_gather(1 - s)
            if wait_w:
                wait_wb(s)
            scale(s)
            issue_wb(g, s)

        issue_idx(0, 0)
        issue_idx(1, 1)
        wait_idx(0)
        issue_gather(0)
        step(0, 0, False, True, True)
        step(1, 1, False, True, True)

        def outer(t, _):
            g = t * 2
            step(g, 0, True, True, True)
            step(g + 1, 1, True, True, True)
            return 0
        lax.fori_loop(1, _N_CHUNKS // 2 - 1, outer, 0)

        step(_N_CHUNKS - 2, 0, True, False, True)
        step(_N_CHUNKS - 1, 1, True, False, False)
        wait_wb(0)
        wait_wb(1)

    return lookup


_lookup = _make_lookup()


def kernel(tokens, table):
    tok_flat = tokens.reshape(_N_IDX)
    out = _lookup(tok_flat, table)
    return out.reshape(B, L, EMB)


# speedup vs baseline: 1.2426x; 1.1122x over previous
import functools
import math

import jax
import jax.numpy as jnp
from jax import lax
from jax.experimental import pallas as pl
from jax.experimental.pallas import tpu as pltpu
from jax.experimental.pallas import tpu_sc as plsc

VOCAB = 1000000
EMB = 64
B = 16384
L = 50
SCALE = math.sqrt(EMB)

_NC = 2
_NS = 16
_NW = _NC * _NS
_BPW = B // _NW
_CHUNK = 256
_SUB = 128
_NSUB = _CHUNK // _SUB
_TPITCH = 257
_UR = 4


def _make_lookup():
    mesh = plsc.VectorSubcoreMesh(core_axis_name="c", subcore_axis_name="s")

    @functools.partial(
        pl.kernel,
        mesh=mesh,
        out_type=jax.ShapeDtypeStruct((L, 8, B // 128, 8, 128), jnp.float32),
        scratch_types=[
            pltpu.VMEM((_BPW, L), jnp.int32),
            pltpu.VMEM((2, _CHUNK), jnp.int32),
            pltpu.VMEM((2, _CHUNK, EMB), jnp.float32),
            pltpu.VMEM((2, EMB, _TPITCH), jnp.float32),
            pltpu.SemaphoreType.DMA,
            pltpu.SemaphoreType.DMA,
            pltpu.SemaphoreType.DMA,
            pltpu.SemaphoreType.DMA,
        ],
        compiler_params=pltpu.CompilerParams(
            use_tc_tiling_on_sc=False, needs_layout_passes=False),
    )
    def lookup(tok_hbm, table_hbm, out_hbm, tok_v, idx_v, rows_in, trans,
               g0, g1, w0, w1):
        gsems = (g0, g1)
        wsems = (w0, w1)
        wid = lax.axis_index("s") * _NC + lax.axis_index("c")
        b0 = wid * _BPW
        iota = lax.iota(jnp.int32, 16)

        def extract_idx(l, half, sdst):
            lcol = jnp.full((16,), l, jnp.int32)

            def ebody(bg, _):
                rows = jnp.full((16,), half * _CHUNK + bg * 16, jnp.int32) + iota
                vals = plsc.load_gather(tok_v, [rows, lcol])
                idx_v[sdst, pl.ds(bg * 16, 16)] = vals
                return 0

            lax.fori_loop(0, _CHUNK // 16, ebody, 0)

        def issue_gather(s):
            for j in range(_NSUB):
                pltpu.async_copy(
                    table_hbm.at[idx_v.at[s].at[pl.ds(j * _SUB, _SUB)]],
                    rows_in.at[s].at[pl.ds(j * _SUB, _SUB)],
                    gsems[s])

        def wait_gather(s):
            for j in range(_NSUB):
                pltpu.make_async_copy(
                    table_hbm.at[idx_v.at[s].at[pl.ds(j * _SUB, _SUB)]],
                    rows_in.at[s].at[pl.ds(j * _SUB, _SUB)],
                    gsems[s]).wait()

        def transpose_scale(s):
            tref = trans.at[s]
            rowc = [iota + c * 16 for c in range(EMB // 16)]

            def body(bq, _):
                for db in range(_UR):
                    bb = bq * _UR + db
                    colv = jnp.full((16,), bb, jnp.int32)
                    for c in range(EMB // 16):
                        v = rows_in[s, bb, pl.ds(c * 16, 16)] * SCALE
                        plsc.store_scatter(tref, [rowc[c], colv], v)
                return 0

            lax.fori_loop(0, _CHUNK // _UR, body, 0)

        def issue_wb(l, half, s):
            for tr in range(8):
                for tc2 in range(2):
                    tb = wid * (_BPW // 128) + half * 2 + tc2
                    pltpu.async_copy(
                        trans.at[s].at[pl.ds(tr * 8, 8), pl.ds(tc2 * 128, 128)],
                        out_hbm.at[l, tr, tb],
                        wsems[s])

        def wait_wb(s):
            for tr in range(8):
                for tc2 in range(2):
                    pltpu.make_async_copy(
                        trans.at[s].at[pl.ds(tr * 8, 8), pl.ds(tc2 * 128, 128)],
                        out_hbm.at[0, tr, 0],
                        wsems[s]).wait()

        def step(t, s, wait_w, iss_g):
            wait_gather(s)
            if iss_g:
                extract_idx(t + s, 1 - s, 1 - s)
                issue_gather(1 - s)
            if wait_w:
                wait_wb(s)
            transpose_scale(s)
            issue_wb(t, s, s)

        pltpu.sync_copy(tok_hbm.at[pl.ds(b0, _BPW)], tok_v)
        extract_idx(0, 0, 0)
        issue_gather(0)
        step(0, 0, False, True)
        step(0, 1, False, True)

        def outer(t, _):
            step(t, 0, True, True)
            step(t, 1, True, True)
            return 0
        lax.fori_loop(1, L - 1, outer, 0)

        step(L - 1, 0, True, True)
        step(L - 1, 1, True, False)
        wait_wb(0)
        wait_wb(1)

    return lookup


_lookup = _make_lookup()


def kernel(tokens, table):
    out5 = _lookup(tokens, table)
    return out5.transpose(2, 4, 0, 1, 3).reshape(B, L, EMB)


# speedup vs baseline: 1.2459x; 1.0027x over previous
import functools
import math

import jax
import jax.numpy as jnp
from jax import lax
from jax.experimental import pallas as pl
from jax.experimental.pallas import tpu as pltpu
from jax.experimental.pallas import tpu_sc as plsc

VOCAB = 1000000
EMB = 64
B = 16384
L = 50
SCALE = math.sqrt(EMB)

_NC = 2
_NS = 16
_NW = _NC * _NS
_BPW = B // _NW
_CHUNK = 256
_SUB = 128
_NSUB = _CHUNK // _SUB
_TPITCH = 257
_UR = 8


def _make_lookup():
    mesh = plsc.VectorSubcoreMesh(core_axis_name="c", subcore_axis_name="s")

    @functools.partial(
        pl.kernel,
        mesh=mesh,
        out_type=jax.ShapeDtypeStruct((L, 8, B // 128, 8, 128), jnp.float32),
        scratch_types=[
            pltpu.VMEM((_BPW, L), jnp.int32),
            pltpu.VMEM((2, _CHUNK), jnp.int32),
            pltpu.VMEM((2, _CHUNK, EMB), jnp.float32),
            pltpu.VMEM((2, EMB, _TPITCH), jnp.float32),
            pltpu.SemaphoreType.DMA,
            pltpu.SemaphoreType.DMA,
            pltpu.SemaphoreType.DMA,
            pltpu.SemaphoreType.DMA,
        ],
        compiler_params=pltpu.CompilerParams(
            use_tc_tiling_on_sc=False, needs_layout_passes=False),
    )
    def lookup(tok_hbm, table_hbm, out_hbm, tok_v, idx_v, rows_in, trans,
               g0, g1, w0, w1):
        gsems = (g0, g1)
        wsems = (w0, w1)
        wid = lax.axis_index("s") * _NC + lax.axis_index("c")
        b0 = wid * _BPW
        iota = lax.iota(jnp.int32, 16)

        def extract_idx(l, half, sdst):
            lcol = jnp.full((16,), l, jnp.int32)

            def ebody(bg, _):
                rows = jnp.full((16,), half * _CHUNK + bg * 16, jnp.int32) + iota
                vals = plsc.load_gather(tok_v, [rows, lcol])
                idx_v[sdst, pl.ds(bg * 16, 16)] = vals
                return 0

            lax.fori_loop(0, _CHUNK // 16, ebody, 0)

        def issue_gather(s):
            for j in range(_NSUB):
                pltpu.async_copy(
                    table_hbm.at[idx_v.at[s].at[pl.ds(j * _SUB, _SUB)]],
                    rows_in.at[s].at[pl.ds(j * _SUB, _SUB)],
                    gsems[s])

        def wait_gather(s):
            pltpu.make_async_copy(
                table_hbm.at[idx_v.at[s]],
                rows_in.at[s],
                gsems[s]).wait()

        def transpose_scale(s):
            tref = trans.at[s]
            rowc = [iota + c * 16 for c in range(EMB // 16)]

            def body(bq, _):
                for db in range(_UR):
                    bb = bq * _UR + db
                    colv = jnp.full((16,), bb, jnp.int32)
                    for c in range(EMB // 16):
                        v = rows_in[s, bb, pl.ds(c * 16, 16)] * SCALE
                        plsc.store_scatter(tref, [rowc[c], colv], v)
                return 0

            lax.fori_loop(0, _CHUNK // _UR, body, 0)

        def issue_wb(l, half, s):
            for tr in range(8):
                for tc2 in range(2):
                    tb = wid * (_BPW // 128) + half * 2 + tc2
                    pltpu.async_copy(
                        trans.at[s].at[pl.ds(tr * 8, 8), pl.ds(tc2 * 128, 128)],
                        out_hbm.at[l, tr, tb],
                        wsems[s])

        def wait_wb(s):
            pltpu.make_async_copy(
                rows_in.at[s],
                table_hbm.at[pl.ds(0, _CHUNK)],
                wsems[s]).wait()

        def step(t, s, wait_w, iss_g):
            wait_gather(s)
            if iss_g:
                extract_idx(t + s, 1 - s, 1 - s)
                issue_gather(1 - s)
            if wait_w:
                wait_wb(s)
            transpose_scale(s)
            issue_wb(t, s, s)

        pltpu.sync_copy(tok_hbm.at[pl.ds(b0, _BPW)], tok_v)
        extract_idx(0, 0, 0)
        issue_gather(0)
        step(0, 0, False, True)
        step(0, 1, False, True)

        def outer(t, _):
            step(t, 0, True, True)
            step(t, 1, True, True)
            return 0
        lax.fori_loop(1, L - 1, outer, 0)

        step(L - 1, 0, True, True)
        step(L - 1, 1, True, False)
        wait_wb(0)
        wait_wb(1)

    return lookup


_lookup = _make_lookup()


def kernel(tokens, table):
    out5 = _lookup(tokens, table)
    return out5.transpose(2, 4, 0, 1, 3).reshape(B, L, EMB)


# speedup vs baseline: 1.8196x; 1.4604x over previous
import functools
import math

import jax
import jax.numpy as jnp
from jax import lax
from jax.experimental import pallas as pl
from jax.experimental.pallas import tpu as pltpu
from jax.experimental.pallas import tpu_sc as plsc

VOCAB = 1000000
EMB = 64
B = 16384
L = 50
SCALE = math.sqrt(EMB)

_NC = 2
_NS = 16
_NW = _NC * _NS
_BPW = B // _NW
_CHUNK = 256
_SUB = 128
_NSUB = _CHUNK // _SUB
_TPITCH = 257


def _make_lookup():
    mesh = plsc.VectorSubcoreMesh(core_axis_name="c", subcore_axis_name="s")

    @functools.partial(
        pl.kernel,
        mesh=mesh,
        out_type=jax.ShapeDtypeStruct((L, 8, B // 128, 8, 128), jnp.float32),
        scratch_types=[
            pltpu.VMEM((_BPW, L), jnp.int32),
            pltpu.VMEM((2, _CHUNK), jnp.int32),
            pltpu.VMEM((2, _CHUNK, EMB), jnp.float32),
            pltpu.VMEM((2, EMB, _TPITCH), jnp.float32),
            pltpu.SemaphoreType.DMA,
            pltpu.SemaphoreType.DMA,
            pltpu.SemaphoreType.DMA,
            pltpu.SemaphoreType.DMA,
        ],
        compiler_params=pltpu.CompilerParams(
            use_tc_tiling_on_sc=False, needs_layout_passes=False),
    )
    def lookup(tok_hbm, table_hbm, out_hbm, tok_v, idx_v, rows_in, trans,
               g0, g1, w0, w1):
        gsems = (g0, g1)
        wsems = (w0, w1)
        wid = lax.axis_index("s") * _NC + lax.axis_index("c")
        b0 = wid * _BPW
        iota = lax.iota(jnp.int32, 16)

        def extract_idx(l, half, sdst):
            lcol = jnp.full((16,), l, jnp.int32)

            @plsc.parallel_loop(0, _CHUNK // 16, unroll=4)
            def ebody(bg):
                rows = jnp.full((16,), half * _CHUNK, jnp.int32) + bg * 16 + iota
                vals = plsc.load_gather(tok_v, [rows, lcol])
                idx_v[sdst, pl.ds(bg * 16, 16)] = vals

        def issue_gather(s):
            for j in range(_NSUB):
                pltpu.async_copy(
                    table_hbm.at[idx_v.at[s].at[pl.ds(j * _SUB, _SUB)]],
                    rows_in.at[s].at[pl.ds(j * _SUB, _SUB)],
                    gsems[s])

        def wait_gather(s):
            pltpu.make_async_copy(
                table_hbm.at[idx_v.at[s]],
                rows_in.at[s],
                gsems[s]).wait()

        def transpose_scale(s):
            tref = trans.at[s]
            rowc = [iota + c * 16 for c in range(EMB // 16)]

            @plsc.parallel_loop(0, _CHUNK, unroll=8)
            def body(bb):
                colv = jnp.full((16,), bb, jnp.int32)
                for c in range(EMB // 16):
                    v = rows_in[s, bb, pl.ds(c * 16, 16)] * SCALE
                    plsc.store_scatter(tref, [rowc[c], colv], v)

        def issue_wb(l, half, s):
            for tr in range(8):
                for tc2 in range(2):
                    tb = wid * (_BPW // 128) + half * 2 + tc2
                    pltpu.async_copy(
                        trans.at[s].at[pl.ds(tr * 8, 8), pl.ds(tc2 * 128, 128)],
                        out_hbm.at[l, tr, tb],
                        wsems[s])

        def wait_wb(s):
            pltpu.make_async_copy(
                rows_in.at[s],
                table_hbm.at[pl.ds(0, _CHUNK)],
                wsems[s]).wait()

        def step(t, s, wait_w, iss_g):
            wait_gather(s)
            if iss_g:
                extract_idx(t + s, 1 - s, 1 - s)
                issue_gather(1 - s)
            if wait_w:
                wait_wb(s)
            transpose_scale(s)
            issue_wb(t, s, s)

        pltpu.sync_copy(tok_hbm.at[pl.ds(b0, _BPW)], tok_v)
        extract_idx(0, 0, 0)
        issue_gather(0)
        step(0, 0, False, True)
        step(0, 1, False, True)

        def outer(t, _):
            step(t, 0, True, True)
            step(t, 1, True, True)
            return 0
        lax.fori_loop(1, L - 1, outer, 0)

        step(L - 1, 0, True, True)
        step(L - 1, 1, True, False)
        wait_wb(0)
        wait_wb(1)

    return lookup


_lookup = _make_lookup()


def kernel(tokens, table):
    out5 = _lookup(tokens, table)
    return out5.transpose(2, 4, 0, 1, 3).reshape(B, L, EMB)


# speedup vs baseline: 2.3350x; 1.2833x over previous
import functools
import math

import jax
import jax.numpy as jnp
from jax import lax
from jax.experimental import pallas as pl
from jax.experimental.pallas import tpu as pltpu
from jax.experimental.pallas import tpu_sc as plsc

VOCAB = 1000000
EMB = 64
B = 16384
L = 50
SCALE = math.sqrt(EMB)

_NC = 2
_NS = 16
_NW = _NC * _NS
_BPW = B // _NW
_CHUNK = 256
_SUB = 128
_NSUB = _CHUNK // _SUB
_TPITCH = 257


_FBV = 8192
_FBR = _FBV // 2
_FGRID = -(-VOCAB // _FBV)


def _fmt_body(x_ref, o_ref):
    y = x_ref[...].T * SCALE
    z = y.reshape(_FBR, 2, EMB)
    o_ref[...] = jnp.concatenate([z[:, 0, :], z[:, 1, :]], axis=1)


def _make_format():
    return pl.pallas_call(
        _fmt_body,
        grid=(_FGRID,),
        in_specs=[pl.BlockSpec((EMB, _FBV), lambda i: (0, i))],
        out_specs=pl.BlockSpec((_FBR, 128), lambda i: (i, 0)),
        out_shape=jax.ShapeDtypeStruct((VOCAB // 2, 128), jnp.float32),
    )


def _make_lookup():
    mesh = plsc.VectorSubcoreMesh(core_axis_name="c", subcore_axis_name="s")

    @functools.partial(
        pl.kernel,
        mesh=mesh,
        out_type=jax.ShapeDtypeStruct((L, 8, B // 128, 8, 128), jnp.float32),
        scratch_types=[
            pltpu.VMEM((_BPW, L), jnp.int32),
            pltpu.VMEM((2, _CHUNK), jnp.int32),
            pltpu.VMEM((2, _CHUNK, EMB), jnp.float32),
            pltpu.VMEM((2, EMB, _TPITCH), jnp.float32),
            pltpu.SemaphoreType.DMA,
            pltpu.SemaphoreType.DMA,
            pltpu.SemaphoreType.DMA,
            pltpu.SemaphoreType.DMA,
        ],
        compiler_params=pltpu.CompilerParams(
            use_tc_tiling_on_sc=False, needs_layout_passes=False),
    )
    def lookup(tok_hbm, table_hbm, out_hbm, tok_v, idx_v, rows_in, trans,
               g0, g1, w0, w1):
        gsems = (g0, g1)
        wsems = (w0, w1)
        wid = lax.axis_index("s") * _NC + lax.axis_index("c")
        b0 = wid * _BPW
        iota = lax.iota(jnp.int32, 16)

        def extract_idx(l, half, sdst):
            lcol = jnp.full((16,), l, jnp.int32)

            @plsc.parallel_loop(0, _CHUNK // 16, unroll=4)
            def ebody(bg):
                rows = jnp.full((16,), half * _CHUNK, jnp.int32) + bg * 16 + iota
                vals = plsc.load_gather(tok_v, [rows, lcol])
                idx_v[sdst, pl.ds(bg * 16, 16)] = vals

        def issue_gather(s):
            for j in range(_NSUB):
                pltpu.async_copy(
                    table_hbm.at[idx_v.at[s].at[pl.ds(j * _SUB, _SUB)]],
                    rows_in.at[s].at[pl.ds(j * _SUB, _SUB)],
                    gsems[s])

        def wait_gather(s):
            pltpu.make_async_copy(
                table_hbm.at[idx_v.at[s]],
                rows_in.at[s],
                gsems[s]).wait()

        def transpose_scale(s):
            tref = trans.at[s]
            rowc = [iota + c * 16 for c in range(EMB // 16)]

            @plsc.parallel_loop(0, _CHUNK, unroll=8)
            def body(bb):
                colv = jnp.full((16,), bb, jnp.int32)
                for c in range(EMB // 16):
                    v = rows_in[s, bb, pl.ds(c * 16, 16)]
                    plsc.store_scatter(tref, [rowc[c], colv], v)

        def issue_wb(l, half, s):
            for tr in range(8):
                for tc2 in range(2):
                    tb = wid * (_BPW // 128) + half * 2 + tc2
                    pltpu.async_copy(
                        trans.at[s].at[pl.ds(tr * 8, 8), pl.ds(tc2 * 128, 128)],
                        out_hbm.at[l, tr, tb],
                        wsems[s])

        def wait_wb(s):
            pltpu.make_async_copy(
                rows_in.at[s],
                table_hbm.at[pl.ds(0, _CHUNK)],
                wsems[s]).wait()

        def step(t, s, wait_w, iss_g):
            wait_gather(s)
            if iss_g:
                extract_idx(t + s, 1 - s, 1 - s)
                issue_gather(1 - s)
            if wait_w:
                wait_wb(s)
            transpose_scale(s)
            issue_wb(t, s, s)

        pltpu.sync_copy(tok_hbm.at[pl.ds(b0, _BPW)], tok_v)
        extract_idx(0, 0, 0)
        issue_gather(0)
        step(0, 0, False, True)
        step(0, 1, False, True)

        def outer(t, _):
            step(t, 0, True, True)
            step(t, 1, True, True)
            return 0
        lax.fori_loop(1, L - 1, outer, 0)

        step(L - 1, 0, True, True)
        step(L - 1, 1, True, False)
        wait_wb(0)
        wait_wb(1)

    return lookup


_lookup = _make_lookup()
_format = _make_format()


def kernel(tokens, table):
    table_rm = _format(table.T).reshape(VOCAB, EMB)
    out5 = _lookup(tokens, table_rm)
    return out5.transpose(2, 4, 0, 1, 3).reshape(B, L, EMB)


# speedup vs baseline: 2.3416x; 1.0028x over previous
import functools
import math

import jax
import jax.numpy as jnp
from jax import lax
from jax.experimental import pallas as pl
from jax.experimental.pallas import tpu as pltpu
from jax.experimental.pallas import tpu_sc as plsc

VOCAB = 1000000
EMB = 64
B = 16384
L = 50
SCALE = math.sqrt(EMB)

_NC = 2
_NS = 16
_NW = _NC * _NS
_BPW = B // _NW
_CHUNK = 256
_SUB = 128
_NSUB = _CHUNK // _SUB
_TPITCH = 257


_FBV = 8192
_FBR = _FBV // 2
_FGRID = -(-VOCAB // _FBV)


def _fmt_body(x_ref, o_ref):
    y = x_ref[...].T * SCALE
    z = y.reshape(_FBR, 2, EMB)
    o_ref[:, 0:EMB] = z[:, 0, :]
    o_ref[:, EMB:128] = z[:, 1, :]


def _make_format():
    return pl.pallas_call(
        _fmt_body,
        grid=(_FGRID,),
        in_specs=[pl.BlockSpec((EMB, _FBV), lambda i: (0, i))],
        out_specs=pl.BlockSpec((_FBR, 128), lambda i: (i, 0)),
        out_shape=jax.ShapeDtypeStruct((VOCAB // 2, 128), jnp.float32),
    )


def _make_lookup():
    mesh = plsc.VectorSubcoreMesh(core_axis_name="c", subcore_axis_name="s")

    @functools.partial(
        pl.kernel,
        mesh=mesh,
        out_type=jax.ShapeDtypeStruct((L, 8, B // 128, 8, 128), jnp.float32),
        scratch_types=[
            pltpu.VMEM((_BPW, L), jnp.int32),
            pltpu.VMEM((2, _CHUNK), jnp.int32),
            pltpu.VMEM((2, _CHUNK, EMB), jnp.float32),
            pltpu.VMEM((2, EMB, _TPITCH), jnp.float32),
            pltpu.SemaphoreType.DMA,
            pltpu.SemaphoreType.DMA,
            pltpu.SemaphoreType.DMA,
            pltpu.SemaphoreType.DMA,
        ],
        compiler_params=pltpu.CompilerParams(
            use_tc_tiling_on_sc=False, needs_layout_passes=False),
    )
    def lookup(tok_hbm, table_hbm, out_hbm, tok_v, idx_v, rows_in, trans,
               g0, g1, w0, w1):
        gsems = (g0, g1)
        wsems = (w0, w1)
        wid = lax.axis_index("s") * _NC + lax.axis_index("c")
        b0 = wid * _BPW
        iota = lax.iota(jnp.int32, 16)

        def extract_idx(l, half, sdst):
            lcol = jnp.full((16,), l, jnp.int32)

            @plsc.parallel_loop(0, _CHUNK // 16, unroll=4)
            def ebody(bg):
                rows = jnp.full((16,), half * _CHUNK, jnp.int32) + bg * 16 + iota
                vals = plsc.load_gather(tok_v, [rows, lcol])
                idx_v[sdst, pl.ds(bg * 16, 16)] = vals

        def issue_gather(s):
            for j in range(_NSUB):
                pltpu.async_copy(
                    table_hbm.at[idx_v.at[s].at[pl.ds(j * _SUB, _SUB)]],
                    rows_in.at[s].at[pl.ds(j * _SUB, _SUB)],
                    gsems[s])

        def wait_gather(s):
            pltpu.make_async_copy(
                table_hbm.at[idx_v.at[s]],
                rows_in.at[s],
                gsems[s]).wait()

        def transpose_scale(s):
            tref = trans.at[s]
            rowc = [iota + c * 16 for c in range(EMB // 16)]

            @plsc.parallel_loop(0, _CHUNK, unroll=8)
            def body(bb):
                colv = jnp.full((16,), bb, jnp.int32)
                for c in range(EMB // 16):
                    v = rows_in[s, bb, pl.ds(c * 16, 16)]
                    plsc.store_scatter(tref, [rowc[c], colv], v)

        def issue_wb(l, half, s):
            for tr in range(8):
                for tc2 in range(2):
                    tb = wid * (_BPW // 128) + half * 2 + tc2
                    pltpu.async_copy(
                        trans.at[s].at[pl.ds(tr * 8, 8), pl.ds(tc2 * 128, 128)],
                        out_hbm.at[l, tr, tb],
                        wsems[s])

        def wait_wb(s):
            pltpu.make_async_copy(
                rows_in.at[s],
                table_hbm.at[pl.ds(0, _CHUNK)],
                wsems[s]).wait()

        def step(t, s, wait_w, iss_g):
            wait_gather(s)
            if iss_g:
                extract_idx(t + s, 1 - s, 1 - s)
                issue_gather(1 - s)
            if wait_w:
                wait_wb(s)
            transpose_scale(s)
            issue_wb(t, s, s)

        pltpu.sync_copy(tok_hbm.at[pl.ds(b0, _BPW)], tok_v)
        extract_idx(0, 0, 0)
        issue_gather(0)
        step(0, 0, False, True)
        step(0, 1, False, True)

        def outer(t, _):
            step(t, 0, True, True)
            step(t, 1, True, True)
            return 0
        lax.fori_loop(1, L - 1, outer, 0)

        step(L - 1, 0, True, True)
        step(L - 1, 1, True, False)
        wait_wb(0)
        wait_wb(1)

    return lookup


_lookup = _make_lookup()
_format = _make_format()


def kernel(tokens, table):
    table_rm = _format(table.T).reshape(VOCAB, EMB)
    out5 = _lookup(tokens, table_rm)
    return out5.transpose(2, 4, 0, 1, 3).reshape(B, L, EMB)
